# baseline (speedup 1.0000x reference)
import jax
import jax.numpy as jnp
from jax import lax
from jax.experimental import pallas as pl
from jax.experimental.pallas import tpu as pltpu
from jax.experimental.pallas import tpu_sc as plsc

N = 10000
NP = 10240
E = 320000
D = 128
NC = 2
NS = 16
NW = NC * NS
EPT = E // NW
K = 80
CH = EPT // K
STRIPE = NP // NS
K2 = 128
CH2 = 79

_MESH = plsc.VectorSubcoreMesh(
    core_axis_name="c", subcore_axis_name="s", num_cores=NC, num_subcores=NS)
_SC_PARAMS = pltpu.CompilerParams(use_tc_tiling_on_sc=False)


def _zero_vmem_rows(ref, nrows, ncols):
  z = jnp.zeros((16,), jnp.float32)

  def body(i, carry):
    r = i // (ncols // 16)
    col = (i % (ncols // 16)) * 16
    ref[r, pl.ds(col, 16)] = z
    return carry

  lax.fori_loop(0, nrows * (ncols // 16), body, 0)


def _fill_ones_rows(ref, nrows):
  o = jnp.ones((16,), jnp.float32)

  def body(i, carry):
    ref[i, pl.ds(0, 16)] = o
    return carry

  lax.fori_loop(0, nrows, body, 0)


def _sc_counts_body(si3, so3, do3, c_out,
                    cacc0, cacc1, cacc2, cbounce, onesv, didx_all, ssem):
  c = lax.axis_index("c")
  s = lax.axis_index("s")
  blk = c * NS + s
  stripe = s * STRIPE

  _zero_vmem_rows(cbounce, STRIPE, 16)
  _fill_ones_rows(onesv, K)

  pltpu.sync_copy(cbounce, cacc0.at[pl.ds(stripe, STRIPE)])
  pltpu.sync_copy(cbounce, cacc1.at[pl.ds(stripe, STRIPE)])
  pltpu.sync_copy(cbounce, cacc2.at[pl.ds(stripe, STRIPE)])
  plsc.subcore_barrier()

  B = 5

  for idx3, cacc in ((so3, cacc0), (do3, cacc1), (si3, cacc2)):
    pltpu.sync_copy(idx3.at[blk], didx_all)

    def cbody(g, carry, cacc=cacc):
      for b in range(B):
        pltpu.async_copy(onesv, cacc.at[didx_all.at[g * B + b]], ssem,
                         add=True)
      for b in range(B):
        pltpu.make_async_copy(onesv, cacc.at[didx_all.at[g * B + b]],
                              ssem).wait()
      return carry

    lax.fori_loop(0, CH // B, cbody, 0)
  plsc.subcore_barrier()

  for j, cacc in enumerate((cacc0, cacc1, cacc2)):
    pltpu.sync_copy(cacc.at[pl.ds(stripe, STRIPE)], cbounce)
    pltpu.sync_copy(
        cbounce, c_out.at[pl.ds((j * NC + c) * NP + stripe, STRIPE)])


@jax.jit
def _sc_counts(dst_rev3, src_of3, dst_of3):
  return pl.kernel(
      _sc_counts_body,
      out_type=jax.ShapeDtypeStruct((3 * NC * NP, 16), jnp.float32),
      mesh=_MESH,
      compiler_params=_SC_PARAMS,
      scratch_types=[
          pltpu.VMEM_SHARED((NP, 16), jnp.float32),
          pltpu.VMEM_SHARED((NP, 16), jnp.float32),
          pltpu.VMEM_SHARED((NP, 16), jnp.float32),
          pltpu.VMEM((STRIPE, 16), jnp.float32),
          pltpu.VMEM((K, 16), jnp.float32),
          pltpu.VMEM((CH, K), jnp.int32),
          pltpu.SemaphoreType.DMA,
      ],
  )(dst_rev3, src_of3, dst_of3)


def _sc_spmm_body(x_hbm, si_hbm, di3, s_out,
                  sacc, didx_all, sidx0, sidx1, rows,
                  gsem0, gsem1, isem0, isem1):
  c = lax.axis_index("c")
  s = lax.axis_index("s")
  blk = c * NS + s
  stripe = s * STRIPE
  ibase = blk * (CH2 * K2)

  _zero_vmem_rows(rows.at[0], K2, D)
  for j in range(STRIPE // K2):
    pltpu.sync_copy(rows.at[0], sacc.at[pl.ds(stripe + j * K2, K2)])
  pltpu.sync_copy(di3.at[blk], didx_all)
  plsc.subcore_barrier()

  pltpu.sync_copy(si_hbm.at[pl.ds(ibase, K2)], sidx0)
  pltpu.async_copy(x_hbm.at[sidx0], rows.at[0], gsem0)
  pltpu.async_copy(si_hbm.at[pl.ds(ibase + K2, K2)], sidx1, isem1)

  def body(i, carry):
    def step(b, sb, osb, sem, osem, isem, oisem):
      pltpu.make_async_copy(x_hbm.at[sb], rows.at[b], sem).wait()

      @pl.when(i + 2 < CH2)
      def _():
        pltpu.async_copy(si_hbm.at[pl.ds(ibase + (i + 2) * K2, K2)], sb, isem)

      @pl.when(i + 1 < CH2)
      def _():
        pltpu.make_async_copy(
            si_hbm.at[pl.ds(ibase + (i + 1) * K2, K2)], osb, oisem).wait()
        pltpu.async_copy(x_hbm.at[osb], rows.at[1 - b], osem)

      pltpu.sync_copy(rows.at[b], sacc.at[didx_all.at[i]], add=True)

    @pl.when(lax.rem(i, 2) == 0)
    def _():
      step(0, sidx0, sidx1, gsem0, gsem1, isem0, isem1)

    @pl.when(lax.rem(i, 2) == 1)
    def _():
      step(1, sidx1, sidx0, gsem1, gsem0, isem1, isem0)

    return carry

  lax.fori_loop(0, CH2, body, 0)
  plsc.subcore_barrier()

  for j in range(STRIPE // K2):
    pltpu.sync_copy(sacc.at[pl.ds(stripe + j * K2, K2)], rows.at[0])
    pltpu.sync_copy(rows.at[0], s_out.at[pl.ds(c * NP + stripe + j * K2, K2)])


@jax.jit
def _sc_spmm(x, si, di3):
  return pl.kernel(
      _sc_spmm_body,
      out_type=jax.ShapeDtypeStruct((NC * NP, D), jnp.float32),
      mesh=_MESH,
      compiler_params=_SC_PARAMS,
      scratch_types=[
          pltpu.VMEM_SHARED((NP, D), jnp.float32),
          pltpu.VMEM((CH2, K2), jnp.int32),
          pltpu.VMEM((K2,), jnp.int32),
          pltpu.VMEM((K2,), jnp.int32),
          pltpu.VMEM((2, K2, D), jnp.float32),
          pltpu.SemaphoreType.DMA,
          pltpu.SemaphoreType.DMA,
          pltpu.SemaphoreType.DMA,
          pltpu.SemaphoreType.DMA,
      ],
  )(x, si, di3)


BM = 1280


def _tc_mid_body(tp_ref, cp_ref, xr_ref, wn_ref, wr_ref, bs_ref, o_ref):
  t = tp_ref[0] + tp_ref[1]
  cnt = jnp.sum(cp_ref[...], axis=3)
  deg = (cnt[:, 0, :] + cnt[:, 1, :]) * (1.0 / 16.0)
  a = lax.rsqrt(jnp.maximum(deg[0], 1.0))
  inv = 1.0 / jnp.maximum(deg[2], 1.0)
  q = t * inv[:, None]
  h = (jnp.dot(q, wn_ref[...], preferred_element_type=jnp.float32)
       + jnp.dot(xr_ref[...], wr_ref[...], preferred_element_type=jnp.float32)
       + bs_ref[...])
  o_ref[...] = jnp.maximum(h, 0.0) * a[:, None]


@jax.jit
def _tc_mid(tp, cp, xr0p, wn, wr, bs):
  return pl.pallas_call(
      _tc_mid_body,
      grid=(NP // BM,),
      in_specs=[
          pl.BlockSpec((NC, BM, D), lambda i: (0, i, 0)),
          pl.BlockSpec((3, NC, BM, 16), lambda i: (0, 0, i, 0)),
          pl.BlockSpec((BM, D), lambda i: (i, 0)),
          pl.BlockSpec((D, D), lambda i: (0, 0)),
          pl.BlockSpec((D, D), lambda i: (0, 0)),
          pl.BlockSpec((1, D), lambda i: (0, 0)),
      ],
      out_specs=pl.BlockSpec((BM, D), lambda i: (i, 0)),
      out_shape=jax.ShapeDtypeStruct((NP, D), jnp.float32),
  )(tp, cp, xr0p, wn, wr, bs)


def _tc_out_body(sp_ref, cp_ref, wg_ref, bg_ref, wl_ref, bl_ref, o_ref):
  sacc = sp_ref[0] + sp_ref[1]
  cnt = jnp.sum(cp_ref[...], axis=2)
  deg = (cnt[0] + cnt[1]) * (1.0 / 16.0)
  cdeg = lax.rsqrt(jnp.maximum(deg, 1.0))
  p = sacc * cdeg[:, None]
  xo = jnp.maximum(
      jnp.dot(p, wg_ref[...], preferred_element_type=jnp.float32)
      + bg_ref[...], 0.0)
  o_ref[...] = (jnp.dot(xo, wl_ref[...], preferred_element_type=jnp.float32)
                + bl_ref[...])


@jax.jit
def _tc_out(sp, cp1, wg, bg, wl, bl):
  return pl.pallas_call(
      _tc_out_body,
      grid=(NP // BM,),
      in_specs=[
          pl.BlockSpec((NC, BM, D), lambda i: (0, i, 0)),
          pl.BlockSpec((NC, BM, 16), lambda i: (0, i, 0)),
          pl.BlockSpec((D, D), lambda i: (0, 0)),
          pl.BlockSpec((1, D), lambda i: (0, 0)),
          pl.BlockSpec((D, 64), lambda i: (0, 0)),
          pl.BlockSpec((1, 64), lambda i: (0, 0)),
      ],
      out_specs=pl.BlockSpec((BM, 64), lambda i: (i, 0)),
      out_shape=jax.ShapeDtypeStruct((NP, 64), jnp.float32),
  )(sp, cp1, wg, bg, wl, bl)


def kernel(x_retweet, x_original, edge_index_of, edge_index_rev_of,
           W_gcn_0, b_gcn_0, W_sage_nbr_0, W_sage_root_0, b_sage_0,
           W_gcn_1, b_gcn_1, W_sage_nbr_1, W_sage_root_1, b_sage_1,
           W_lin, b_lin):
  src_of = edge_index_of[0].astype(jnp.int32)
  dst_of = edge_index_of[1].astype(jnp.int32)
  src_rev = edge_index_rev_of[0].astype(jnp.int32)
  dst_rev = edge_index_rev_of[1].astype(jnp.int32)

  def pad_src(e):
    e2 = e.reshape(NW, EPT)
    return jnp.pad(e2, ((0, 0), (0, CH2 * K2 - EPT))).reshape(-1)

  def pad_dst(e):
    e2 = e.reshape(NW, EPT)
    e2 = jnp.pad(e2, ((0, 0), (0, CH2 * K2 - EPT)), constant_values=NP - 1)
    return e2.reshape(NW, CH2, K2)

  cflat = _sc_counts(dst_rev.reshape(NW, CH, K), src_of.reshape(NW, CH, K),
                     dst_of.reshape(NW, CH, K))
  cp = cflat.reshape(3, NC, NP, 16)
  tp = _sc_spmm(x_original, pad_src(src_rev),
                pad_dst(dst_rev)).reshape(NC, NP, D)

  xr0p = jnp.pad(x_retweet, ((0, NP - N), (0, 0)))
  xr1s = _tc_mid(tp, cp, xr0p,
                 W_sage_nbr_0, W_sage_root_0, b_sage_0.reshape(1, D))

  sp = _sc_spmm(xr1s, pad_src(src_of),
                pad_dst(dst_of)).reshape(NC, NP, D)

  out = _tc_out(sp, cp[1], W_gcn_1, b_gcn_1.reshape(1, D),
                W_lin, b_lin.reshape(1, 64))
  return out[:N]


# speedup vs baseline: 1.4786x; 1.4786x over previous
import jax
import jax.numpy as jnp
from jax import lax
from jax.experimental import pallas as pl
from jax.experimental.pallas import tpu as pltpu
from jax.experimental.pallas import tpu_sc as plsc

N = 10000
NP = 10240
E = 320000
D = 128
NC = 2
NS = 16
NW = NC * NS
EPT = E // NW
K = 80
CH = EPT // K
STRIPE = NP // NS

_MESH = plsc.VectorSubcoreMesh(
    core_axis_name="c", subcore_axis_name="s", num_cores=NC, num_subcores=NS)
_SC_PARAMS = pltpu.CompilerParams(use_tc_tiling_on_sc=False)


def _zero_vmem_rows(ref, nrows, ncols):
  z = jnp.zeros((16,), jnp.float32)

  def body(i, carry):
    r = i // (ncols // 16)
    col = (i % (ncols // 16)) * 16
    ref[r, pl.ds(col, 16)] = z
    return carry

  lax.fori_loop(0, nrows * (ncols // 16), body, 0)


def _fill_ones_rows(ref, nrows):
  o = jnp.ones((16,), jnp.float32)

  def body(i, carry):
    ref[i, pl.ds(0, 16)] = o
    return carry

  lax.fori_loop(0, nrows, body, 0)


def _sc_counts_body(si3, so3, do3, c_out,
                    cacc0, cacc1, cacc2, cbounce, onesv, didx_all, ssem):
  c = lax.axis_index("c")
  s = lax.axis_index("s")
  blk = c * NS + s
  stripe = s * STRIPE

  _zero_vmem_rows(cbounce, STRIPE, 16)
  _fill_ones_rows(onesv, K)

  pltpu.sync_copy(cbounce, cacc0.at[pl.ds(stripe, STRIPE)])
  pltpu.sync_copy(cbounce, cacc1.at[pl.ds(stripe, STRIPE)])
  pltpu.sync_copy(cbounce, cacc2.at[pl.ds(stripe, STRIPE)])
  plsc.subcore_barrier()

  B = 5

  for idx3, cacc in ((so3, cacc0), (do3, cacc1), (si3, cacc2)):
    pltpu.sync_copy(idx3.at[blk], didx_all)

    def cbody(g, carry, cacc=cacc):
      for b in range(B):
        pltpu.async_copy(onesv, cacc.at[didx_all.at[g * B + b]], ssem,
                         add=True)
      for b in range(B):
        pltpu.make_async_copy(onesv, cacc.at[didx_all.at[g * B + b]],
                              ssem).wait()
      return carry

    lax.fori_loop(0, CH // B, cbody, 0)
  plsc.subcore_barrier()

  for j, cacc in enumerate((cacc0, cacc1, cacc2)):
    pltpu.sync_copy(cacc.at[pl.ds(stripe, STRIPE)], cbounce)
    pltpu.sync_copy(
        cbounce, c_out.at[pl.ds((j * NC + c) * NP + stripe, STRIPE)])


@jax.jit
def _sc_counts(dst_rev3, src_of3, dst_of3):
  return pl.kernel(
      _sc_counts_body,
      out_type=jax.ShapeDtypeStruct((3 * NC * NP, 16), jnp.float32),
      mesh=_MESH,
      compiler_params=_SC_PARAMS,
      scratch_types=[
          pltpu.VMEM_SHARED((NP, 16), jnp.float32),
          pltpu.VMEM_SHARED((NP, 16), jnp.float32),
          pltpu.VMEM_SHARED((NP, 16), jnp.float32),
          pltpu.VMEM((STRIPE, 16), jnp.float32),
          pltpu.VMEM((K, 16), jnp.float32),
          pltpu.VMEM((CH, K), jnp.int32),
          pltpu.SemaphoreType.DMA,
      ],
  )(dst_rev3, src_of3, dst_of3)


def _sc_spmm_body(x_hbm, si3, di3, s_out,
                  sacc, sidx_all, didx_all, rows, gsem0, gsem1):
  c = lax.axis_index("c")
  s = lax.axis_index("s")
  blk = c * NS + s
  stripe = s * STRIPE

  _zero_vmem_rows(rows.at[0], K, D)
  for j in range(STRIPE // K):
    pltpu.sync_copy(rows.at[0], sacc.at[pl.ds(stripe + j * K, K)])
  pltpu.sync_copy(si3.at[blk], sidx_all)
  pltpu.sync_copy(di3.at[blk], didx_all)
  plsc.subcore_barrier()

  pltpu.async_copy(x_hbm.at[sidx_all.at[0]], rows.at[0], gsem0)

  def body(i, carry):
    def step(b, sem, osem):
      pltpu.make_async_copy(x_hbm.at[sidx_all.at[i]], rows.at[b], sem).wait()

      @pl.when(i + 1 < CH)
      def _():
        pltpu.async_copy(x_hbm.at[sidx_all.at[i + 1]], rows.at[1 - b], osem)

      pltpu.sync_copy(rows.at[b], sacc.at[didx_all.at[i]], add=True)

    @pl.when(lax.rem(i, 2) == 0)
    def _():
      step(0, gsem0, gsem1)

    @pl.when(lax.rem(i, 2) == 1)
    def _():
      step(1, gsem1, gsem0)

    return carry

  lax.fori_loop(0, CH, body, 0)
  plsc.subcore_barrier()

  for j in range(STRIPE // K):
    pltpu.sync_copy(sacc.at[pl.ds(stripe + j * K, K)], rows.at[0])
    pltpu.sync_copy(rows.at[0], s_out.at[pl.ds(c * NP + stripe + j * K, K)])


@jax.jit
def _sc_spmm(x, si3, di3):
  return pl.kernel(
      _sc_spmm_body,
      out_type=jax.ShapeDtypeStruct((NC * NP, D), jnp.float32),
      mesh=_MESH,
      compiler_params=_SC_PARAMS,
      scratch_types=[
          pltpu.VMEM_SHARED((NP, D), jnp.float32),
          pltpu.VMEM((CH, K), jnp.int32),
          pltpu.VMEM((CH, K), jnp.int32),
          pltpu.VMEM((2, K, D), jnp.float32),
          pltpu.SemaphoreType.DMA,
          pltpu.SemaphoreType.DMA,
      ],
  )(x, si3, di3)


BM = 1280


def _tc_mid_body(tp_ref, cp_ref, xr_ref, wn_ref, wr_ref, bs_ref, o_ref):
  t = tp_ref[0] + tp_ref[1]
  cnt = jnp.sum(cp_ref[...], axis=3)
  deg = (cnt[:, 0, :] + cnt[:, 1, :]) * (1.0 / 16.0)
  a = lax.rsqrt(jnp.maximum(deg[0], 1.0))
  inv = 1.0 / jnp.maximum(deg[2], 1.0)
  q = t * inv[:, None]
  h = (jnp.dot(q, wn_ref[...], preferred_element_type=jnp.float32)
       + jnp.dot(xr_ref[...], wr_ref[...], preferred_element_type=jnp.float32)
       + bs_ref[...])
  o_ref[...] = jnp.maximum(h, 0.0) * a[:, None]


@jax.jit
def _tc_mid(tp, cp, xr0p, wn, wr, bs):
  return pl.pallas_call(
      _tc_mid_body,
      grid=(NP // BM,),
      in_specs=[
          pl.BlockSpec((NC, BM, D), lambda i: (0, i, 0)),
          pl.BlockSpec((3, NC, BM, 16), lambda i: (0, 0, i, 0)),
          pl.BlockSpec((BM, D), lambda i: (i, 0)),
          pl.BlockSpec((D, D), lambda i: (0, 0)),
          pl.BlockSpec((D, D), lambda i: (0, 0)),
          pl.BlockSpec((1, D), lambda i: (0, 0)),
      ],
      out_specs=pl.BlockSpec((BM, D), lambda i: (i, 0)),
      out_shape=jax.ShapeDtypeStruct((NP, D), jnp.float32),
  )(tp, cp, xr0p, wn, wr, bs)


def _tc_out_body(sp_ref, cp_ref, wg_ref, bg_ref, wl_ref, bl_ref, o_ref):
  sacc = sp_ref[0] + sp_ref[1]
  cnt = jnp.sum(cp_ref[...], axis=2)
  deg = (cnt[0] + cnt[1]) * (1.0 / 16.0)
  cdeg = lax.rsqrt(jnp.maximum(deg, 1.0))
  p = sacc * cdeg[:, None]
  xo = jnp.maximum(
      jnp.dot(p, wg_ref[...], preferred_element_type=jnp.float32)
      + bg_ref[...], 0.0)
  o_ref[...] = (jnp.dot(xo, wl_ref[...], preferred_element_type=jnp.float32)
                + bl_ref[...])


@jax.jit
def _tc_out(sp, cp1, wg, bg, wl, bl):
  return pl.pallas_call(
      _tc_out_body,
      grid=(NP // BM,),
      in_specs=[
          pl.BlockSpec((NC, BM, D), lambda i: (0, i, 0)),
          pl.BlockSpec((NC, BM, 16), lambda i: (0, i, 0)),
          pl.BlockSpec((D, D), lambda i: (0, 0)),
          pl.BlockSpec((1, D), lambda i: (0, 0)),
          pl.BlockSpec((D, 64), lambda i: (0, 0)),
          pl.BlockSpec((1, 64), lambda i: (0, 0)),
      ],
      out_specs=pl.BlockSpec((BM, 64), lambda i: (i, 0)),
      out_shape=jax.ShapeDtypeStruct((NP, 64), jnp.float32),
  )(sp, cp1, wg, bg, wl, bl)


def kernel(x_retweet, x_original, edge_index_of, edge_index_rev_of,
           W_gcn_0, b_gcn_0, W_sage_nbr_0, W_sage_root_0, b_sage_0,
           W_gcn_1, b_gcn_1, W_sage_nbr_1, W_sage_root_1, b_sage_1,
           W_lin, b_lin):
  src_of = edge_index_of[0].astype(jnp.int32).reshape(NW, CH, K)
  dst_of = edge_index_of[1].astype(jnp.int32).reshape(NW, CH, K)
  src_rev = edge_index_rev_of[0].astype(jnp.int32).reshape(NW, CH, K)
  dst_rev = edge_index_rev_of[1].astype(jnp.int32).reshape(NW, CH, K)

  cflat = _sc_counts(dst_rev, src_of, dst_of)
  cp = cflat.reshape(3, NC, NP, 16)
  tp = _sc_spmm(x_original, src_rev, dst_rev).reshape(NC, NP, D)

  xr0p = jnp.pad(x_retweet, ((0, NP - N), (0, 0)))
  xr1s = _tc_mid(tp, cp, xr0p,
                 W_sage_nbr_0, W_sage_root_0, b_sage_0.reshape(1, D))

  sp = _sc_spmm(xr1s, src_of, dst_of).reshape(NC, NP, D)

  out = _tc_out(sp, cp[1], W_gcn_1, b_gcn_1.reshape(1, D),
                W_lin, b_lin.reshape(1, 64))
  return out[:N]
